# baseline (speedup 1.0000x reference)
import functools

import jax
import jax.numpy as jnp
from jax import lax
from jax.experimental import pallas as pl
from jax.experimental.pallas import tpu as pltpu
from jax.experimental.pallas import tpu_sc as plsc

VOCAB = 1000000
EMBED_DIM = 32
BATCH = 4096
TIME = 200
NUM_IDX = BATCH * TIME
NC, NS = 2, 16
NW = NC * NS
PER_W = NUM_IDX // NW
CHUNK = 512
NCH = PER_W // CHUNK
PITCH = 33
NLB = CHUNK // 128

P_SHAPE = (TIME, EMBED_DIM // 8, BATCH // 128, 8, 128)


def _body(idx_hbm, tab_hbm, out_hbm, idx_v0, idx_v1, g_v0, g_v1, gp_v,
          s_v0, s_v1, gsem0, gsem1, osem0, osem1):
    wid = lax.axis_index("s") * NC + lax.axis_index("c")
    base = wid * PER_W
    idx_v = (idx_v0, idx_v1)
    g_v = (g_v0, g_v1)
    s_v = (s_v0, s_v1)
    gsems = (gsem0, gsem1)
    osems = (osem0, osem1)
    iota = lax.iota(jnp.int32, 16)
    zeros = jnp.zeros((16,), jnp.int32)

    def load_idx(j, b):
        pltpu.sync_copy(idx_hbm.at[pl.ds(base + j * CHUNK, CHUNK)], idx_v[b])

    def start_gather(b):
        return pltpu.async_copy(tab_hbm.at[idx_v[b]], g_v[b], gsems[b])

    cols = [jnp.full((16,), e, jnp.int32) for e in range(EMBED_DIM)]

    def transpose(b):
        S = s_v[b]
        Gr = g_v[b]

        def p1(r8, _):
            for d in range(8):
                r = 8 * r8 + d
                for h in range(2):
                    gp_v[r, pl.ds(16 * h, 16)] = Gr[r, pl.ds(16 * h, 16)]
            return 0

        lax.fori_loop(0, CHUNK // 8, p1, 0, unroll=False)
        G = gp_v

        def p2(m, _):
            lb = m // 8
            k = m % 8
            rows = 16 * m + iota
            for e in range(EMBED_DIM):
                v = plsc.load_gather(G, [rows, cols[e]])
                S[e // 8, lb, e % 8, pl.ds(16 * k, 16)] = v
            return 0

        lax.fori_loop(0, NLB * 8, p2, 0, unroll=False)

    def start_store(j, b):
        flat = base + j * CHUNK
        t = flat // BATCH
        lb0 = (flat % BATCH) // 128
        return pltpu.async_copy(
            s_v[b], out_hbm.at[t, :, pl.ds(lb0, NLB)], osems[b]
        )

    def wait_gather(b):
        pltpu.make_async_copy(tab_hbm.at[idx_v[b]], g_v[b], gsems[b]).wait()

    def wait_store(b):
        pltpu.make_async_copy(
            s_v[b], out_hbm.at[0, :, pl.ds(0, NLB)], osems[b]
        ).wait()

    load_idx(0, 0)
    start_gather(0)
    load_idx(1, 1)
    start_gather(1)
    wait_gather(0)
    transpose(0)
    start_store(0, 0)
    load_idx(2, 0)
    start_gather(0)
    wait_gather(1)
    transpose(1)
    start_store(1, 1)

    @pl.loop(1, NCH // 2)
    def _pair(p):
        jj = 2 * p
        load_idx(jj + 1, 1)
        start_gather(1)
        wait_gather(0)
        wait_store(0)
        transpose(0)
        start_store(jj, 0)

        @pl.when(jj + 2 < NCH)
        def _prefetch():
            load_idx(jj + 2, 0)
            start_gather(0)

        wait_gather(1)
        wait_store(1)
        transpose(1)
        start_store(jj + 1, 1)

    wait_store(0)
    wait_store(1)


def _make_call():
    mesh = plsc.VectorSubcoreMesh(
        core_axis_name="c", subcore_axis_name="s", num_cores=NC, num_subcores=NS
    )
    return pl.kernel(
        _body,
        out_type=jax.ShapeDtypeStruct(P_SHAPE, jnp.float32),
        mesh=mesh,
        scratch_types=[
            pltpu.VMEM((CHUNK,), jnp.int32),
            pltpu.VMEM((CHUNK,), jnp.int32),
            pltpu.VMEM((CHUNK, EMBED_DIM), jnp.float32),
            pltpu.VMEM((CHUNK, EMBED_DIM), jnp.float32),
            pltpu.VMEM((CHUNK, PITCH), jnp.float32),
            pltpu.VMEM((EMBED_DIM // 8, NLB, 8, 128), jnp.float32),
            pltpu.VMEM((EMBED_DIM // 8, NLB, 8, 128), jnp.float32),
            pltpu.SemaphoreType.DMA,
            pltpu.SemaphoreType.DMA,
            pltpu.SemaphoreType.DMA,
            pltpu.SemaphoreType.DMA,
        ],
        compiler_params=pltpu.CompilerParams(
            use_tc_tiling_on_sc=False, needs_layout_passes=False
        ),
    )


@jax.jit
def kernel(items, table):
    idx = items.T.reshape(-1).astype(jnp.int32)
    outP = _make_call()(idx, table)
    return outP.transpose(2, 4, 0, 1, 3).reshape(BATCH, TIME, EMBED_DIM)


# speedup vs baseline: 1.9046x; 1.9046x over previous
import functools

import jax
import jax.numpy as jnp
from jax import lax
from jax.experimental import pallas as pl
from jax.experimental.pallas import tpu as pltpu
from jax.experimental.pallas import tpu_sc as plsc

VOCAB = 1000000
EMBED_DIM = 32
BATCH = 4096
TIME = 200
NUM_IDX = BATCH * TIME
NC, NS = 2, 16
NW = NC * NS
PER_W = NUM_IDX // NW
CHUNK = 512
NCH = PER_W // CHUNK
PAD = 131
NLB = CHUNK // 128

P_SHAPE = (TIME, EMBED_DIM // 8, BATCH // 128, 8, 128)


def _body(idx_hbm, tab_hbm, out_hbm, idx_v0, idx_v1, g_v0, g_v1,
          s_v0, s_v1, gsem0, gsem1, osem0, osem1):
    wid = lax.axis_index("s") * NC + lax.axis_index("c")
    base = wid * PER_W
    idx_v = (idx_v0, idx_v1)
    g_v = (g_v0, g_v1)
    s_v = (s_v0, s_v1)
    gsems = (gsem0, gsem1)
    osems = (osem0, osem1)
    iota = lax.iota(jnp.int32, 16)
    zeros = jnp.zeros((16,), jnp.int32)

    def load_idx(j, b):
        pltpu.sync_copy(idx_hbm.at[pl.ds(base + j * CHUNK, CHUNK)], idx_v[b])

    def start_gather(b):
        return pltpu.async_copy(tab_hbm.at[idx_v[b]], g_v[b], gsems[b])

    se_vs = [lax.iota(jnp.int32, 16) // 8 + 2 * h for h in range(2)]
    er_vs = [lax.iota(jnp.int32, 16) % 8 for _ in range(2)]

    def transpose(b):
        S = s_v[b]
        G = g_v[b]

        def p2(r, _):
            lb = r // 128
            bc = r % 128
            lbv = zeros + lb
            bcv = zeros + bc
            for h in range(2):
                v = G[r, pl.ds(16 * h, 16)]
                plsc.store_scatter(S, [se_vs[h], lbv, er_vs[h], bcv], v)
            return 0

        lax.fori_loop(0, CHUNK, p2, 0, unroll=False)

    def start_store(j, b):
        flat = base + j * CHUNK
        t = flat // BATCH
        lb0 = (flat % BATCH) // 128
        return pltpu.async_copy(
            s_v[b].at[:, :, :, pl.ds(0, 128)],
            out_hbm.at[t, :, pl.ds(lb0, NLB)], osems[b]
        )

    def wait_gather(b):
        pltpu.make_async_copy(tab_hbm.at[idx_v[b]], g_v[b], gsems[b]).wait()

    def wait_store(b):
        pltpu.make_async_copy(
            s_v[b].at[:, :, :, pl.ds(0, 128)],
            out_hbm.at[0, :, pl.ds(0, NLB)], osems[b]
        ).wait()

    load_idx(0, 0)
    start_gather(0)
    load_idx(1, 1)
    start_gather(1)
    wait_gather(0)
    transpose(0)
    start_store(0, 0)
    load_idx(2, 0)
    start_gather(0)
    wait_gather(1)
    transpose(1)
    start_store(1, 1)

    @pl.loop(1, NCH // 2)
    def _pair(p):
        jj = 2 * p
        load_idx(jj + 1, 1)
        start_gather(1)
        wait_gather(0)
        wait_store(0)
        transpose(0)
        start_store(jj, 0)

        @pl.when(jj + 2 < NCH)
        def _prefetch():
            load_idx(jj + 2, 0)
            start_gather(0)

        wait_gather(1)
        wait_store(1)
        transpose(1)
        start_store(jj + 1, 1)

    wait_store(0)
    wait_store(1)


def _make_call():
    mesh = plsc.VectorSubcoreMesh(
        core_axis_name="c", subcore_axis_name="s", num_cores=NC, num_subcores=NS
    )
    return pl.kernel(
        _body,
        out_type=jax.ShapeDtypeStruct(P_SHAPE, jnp.float32),
        mesh=mesh,
        scratch_types=[
            pltpu.VMEM((CHUNK,), jnp.int32),
            pltpu.VMEM((CHUNK,), jnp.int32),
            pltpu.VMEM((CHUNK, EMBED_DIM), jnp.float32),
            pltpu.VMEM((CHUNK, EMBED_DIM), jnp.float32),
            pltpu.VMEM((EMBED_DIM // 8, NLB, 8, PAD), jnp.float32),
            pltpu.VMEM((EMBED_DIM // 8, NLB, 8, PAD), jnp.float32),
            pltpu.SemaphoreType.DMA,
            pltpu.SemaphoreType.DMA,
            pltpu.SemaphoreType.DMA,
            pltpu.SemaphoreType.DMA,
        ],
        compiler_params=pltpu.CompilerParams(
            use_tc_tiling_on_sc=False, needs_layout_passes=False
        ),
    )


@jax.jit
def kernel(items, table):
    idx = items.T.reshape(-1).astype(jnp.int32)
    outP = _make_call()(idx, table)
    return outP.transpose(2, 4, 0, 1, 3).reshape(BATCH, TIME, EMBED_DIM)


# speedup vs baseline: 1.9608x; 1.0295x over previous
import functools

import jax
import jax.numpy as jnp
from jax import lax
from jax.experimental import pallas as pl
from jax.experimental.pallas import tpu as pltpu
from jax.experimental.pallas import tpu_sc as plsc

VOCAB = 1000000
EMBED_DIM = 32
BATCH = 4096
TIME = 200
NUM_IDX = BATCH * TIME
NC, NS = 2, 16
NW = NC * NS
PER_W = NUM_IDX // NW
CHUNK = 512
NCH = PER_W // CHUNK
PAD = 131
NLB = CHUNK // 128

P_SHAPE = (TIME, EMBED_DIM // 8, BATCH // 128, 8, 128)


def _body(idx_hbm, tab_hbm, out_hbm, idx_v0, idx_v1, g_v0, g_v1,
          s_v0, s_v1, gsem0, gsem1, osem0, osem1):
    wid = lax.axis_index("s") * NC + lax.axis_index("c")
    base = wid * PER_W
    idx_v = (idx_v0, idx_v1)
    g_v = (g_v0, g_v1)
    s_v = (s_v0, s_v1)
    gsems = (gsem0, gsem1)
    osems = (osem0, osem1)
    iota = lax.iota(jnp.int32, 16)
    zeros = jnp.zeros((16,), jnp.int32)

    def load_idx(j, b):
        pltpu.sync_copy(idx_hbm.at[pl.ds(base + j * CHUNK, CHUNK)], idx_v[b])

    def start_gather(b):
        return pltpu.async_copy(tab_hbm.at[idx_v[b]], g_v[b], gsems[b])

    se_vs = [lax.iota(jnp.int32, 16) // 8 + 2 * h for h in range(2)]
    er_vs = [lax.iota(jnp.int32, 16) % 8 for _ in range(2)]

    def transpose(b):
        S = s_v[b]
        G = g_v[b]

        def p2(r4, _):
            r0 = 4 * r4
            lb = r0 // 128
            lbv = zeros + lb
            for d in range(4):
                r = r0 + d
                bcv = zeros + (r % 128)
                for h in range(2):
                    v = G[r, pl.ds(16 * h, 16)]
                    plsc.store_scatter(S, [se_vs[h], lbv, er_vs[h], bcv], v)
            return 0

        lax.fori_loop(0, CHUNK // 4, p2, 0, unroll=False)

    def start_store(j, b):
        flat = base + j * CHUNK
        t = flat // BATCH
        lb0 = (flat % BATCH) // 128
        return pltpu.async_copy(
            s_v[b].at[:, :, :, pl.ds(0, 128)],
            out_hbm.at[t, :, pl.ds(lb0, NLB)], osems[b]
        )

    def wait_gather(b):
        pltpu.make_async_copy(tab_hbm.at[idx_v[b]], g_v[b], gsems[b]).wait()

    def wait_store(b):
        pltpu.make_async_copy(
            s_v[b].at[:, :, :, pl.ds(0, 128)],
            out_hbm.at[0, :, pl.ds(0, NLB)], osems[b]
        ).wait()

    load_idx(0, 0)
    start_gather(0)
    load_idx(1, 1)
    start_gather(1)
    wait_gather(0)
    transpose(0)
    start_store(0, 0)
    load_idx(2, 0)
    start_gather(0)
    wait_gather(1)
    transpose(1)
    start_store(1, 1)

    @pl.loop(1, NCH // 2)
    def _pair(p):
        jj = 2 * p
        load_idx(jj + 1, 1)
        start_gather(1)
        wait_gather(0)
        wait_store(0)
        transpose(0)
        start_store(jj, 0)

        @pl.when(jj + 2 < NCH)
        def _prefetch():
            load_idx(jj + 2, 0)
            start_gather(0)

        wait_gather(1)
        wait_store(1)
        transpose(1)
        start_store(jj + 1, 1)

    wait_store(0)
    wait_store(1)


def _make_call():
    mesh = plsc.VectorSubcoreMesh(
        core_axis_name="c", subcore_axis_name="s", num_cores=NC, num_subcores=NS
    )
    return pl.kernel(
        _body,
        out_type=jax.ShapeDtypeStruct(P_SHAPE, jnp.float32),
        mesh=mesh,
        scratch_types=[
            pltpu.VMEM((CHUNK,), jnp.int32),
            pltpu.VMEM((CHUNK,), jnp.int32),
            pltpu.VMEM((CHUNK, EMBED_DIM), jnp.float32),
            pltpu.VMEM((CHUNK, EMBED_DIM), jnp.float32),
            pltpu.VMEM((EMBED_DIM // 8, NLB, 8, PAD), jnp.float32),
            pltpu.VMEM((EMBED_DIM // 8, NLB, 8, PAD), jnp.float32),
            pltpu.SemaphoreType.DMA,
            pltpu.SemaphoreType.DMA,
            pltpu.SemaphoreType.DMA,
            pltpu.SemaphoreType.DMA,
        ],
        compiler_params=pltpu.CompilerParams(
            use_tc_tiling_on_sc=False, needs_layout_passes=False
        ),
    )


@jax.jit
def kernel(items, table):
    idx = items.T.reshape(-1).astype(jnp.int32)
    outP = _make_call()(idx, table)
    return outP.transpose(2, 4, 0, 1, 3).reshape(BATCH, TIME, EMBED_DIM)
